# Initial kernel scaffold; baseline (speedup 1.0000x reference)
import jax
import jax.numpy as jnp
from jax.experimental import pallas as pl


def kernel(features, edge_index, subgraph_adj, subgraph_norm, node_subgraph, node_list, perm, W_gcn, b_gcn, W_disc):
    raise NotImplementedError("write your pallas kernel here")


# speedup vs baseline: 14.1448x; 14.1448x over previous
import functools

import jax
import jax.numpy as jnp
from jax import lax
from jax.experimental import pallas as pl
from jax.experimental.pallas import tpu as pltpu
from jax.experimental.pallas import tpu_sc as plsc

_N = 10000
_E = 320000
_D = 128
_S = 100
_NC = 2
_NS = 16
_NPAD = 10240
_SEG = _NPAD // _NS
_EPT = _E // _NS
_CH = 128
_FULL = _EPT // _CH
_REM = _EPT - _FULL * _CH
_RPT = 632
_RPT_LAST = _N - (_NS - 1) * _RPT
_GCH = 128
_GFULL = _N // _GCH
_GREM = _N - _GFULL * _GCH


def _fori(n, body, lo=0):
    lax.fori_loop(lo, n, lambda i, c: (body(i), c)[1], 0)


_HW = 16


def _sc_pass1_body(dst_hbm, perm_hbm, feat_hbm, deg_out, xperm_out,
                   hist, zerob, onesb, didx, didx2, idx, rows, sem):
    c = lax.axis_index("c")
    s = lax.axis_index("s")

    @pl.when(c == 0)
    def _degree():
        def fill(i):
            zerob[i] = jnp.zeros((_HW,), jnp.float32)
            onesb[i] = jnp.ones((_HW,), jnp.float32)
        _fori(_CH, fill)
        for q in range(_SEG // _CH):
            pltpu.sync_copy(zerob, hist.at[pl.ds(s * _SEG + q * _CH, _CH)])
        plsc.subcore_barrier()
        base = s * _EPT

        def chunk(g):
            off = base + g * _CH
            pltpu.sync_copy(dst_hbm.at[pl.ds(off, _CH)], didx)
            pltpu.sync_copy(onesb, hist.at[didx], add=True)
        _fori(_FULL, chunk)
        off = base + _FULL * _CH
        pltpu.sync_copy(dst_hbm.at[pl.ds(off, _REM)], didx2)
        pltpu.sync_copy(onesb.at[pl.ds(0, _REM)], hist.at[didx2], add=True)
        plsc.subcore_barrier()
        pltpu.sync_copy(hist.at[pl.ds(s * _SEG, _SEG)],
                        deg_out.at[pl.ds(s * _SEG, _SEG)])

    @pl.when(c == 1)
    def _permgather():
        def gather_chunk(off, idxref, rowsref, n):
            pltpu.sync_copy(perm_hbm.at[pl.ds(off, n)], idxref)
            pltpu.async_copy(feat_hbm.at[idxref], rowsref, sem).wait()
            pltpu.sync_copy(rowsref, xperm_out.at[pl.ds(off, n)])

        for k in range((_GFULL + _NS - 1) // _NS):
            g = s + _NS * k

            @pl.when(g < _GFULL)
            def _do():
                gather_chunk(pl.multiple_of(g * _GCH, _GCH), idx, rows, _GCH)

        @pl.when(s == _NS - 1)
        def _rem():
            gather_chunk(_GFULL * _GCH, idx.at[pl.ds(0, _GREM)],
                         rows.at[pl.ds(0, _GREM)], _GREM)


_sc_pass1 = functools.partial(
    pl.kernel,
    out_type=[jax.ShapeDtypeStruct((_NPAD, _HW), jnp.float32),
              jax.ShapeDtypeStruct((_N, _D), jnp.float32)],
    mesh=plsc.VectorSubcoreMesh(core_axis_name="c", subcore_axis_name="s",
                                num_cores=_NC, num_subcores=_NS),
    scratch_types=[
        pltpu.VMEM_SHARED((_NPAD, _HW), jnp.float32),
        pltpu.VMEM((_CH, _HW), jnp.float32),
        pltpu.VMEM((_CH, _HW), jnp.float32),
        pltpu.VMEM((_CH,), jnp.int32),
        pltpu.VMEM((_REM,), jnp.int32),
        pltpu.VMEM((_GCH,), jnp.int32),
        pltpu.VMEM((_GCH, _D), jnp.float32),
        pltpu.SemaphoreType.DMA,
    ],
)(_sc_pass1_body)


def _sc_pass2_body(xs_hbm, src_hbm, dst_hbm, u_out,
                   table, sidx, didx, rows, sidx2, didx2, rows2, sem):
    c = lax.axis_index("c")
    s = lax.axis_index("s")
    r0 = pl.multiple_of(s * _RPT, 8)

    @pl.when(s < _NS - 1)
    def _init_main():
        pltpu.sync_copy(xs_hbm.at[c, pl.ds(r0, _RPT)],
                        table.at[pl.ds(r0, _RPT)])

    @pl.when(s == _NS - 1)
    def _init_last():
        pltpu.sync_copy(xs_hbm.at[c, pl.ds((_NS - 1) * _RPT, _RPT_LAST)],
                        table.at[pl.ds((_NS - 1) * _RPT, _RPT_LAST)])

    plsc.subcore_barrier()
    base = s * _EPT

    def chunk(g):
        off = base + g * _CH
        pltpu.sync_copy(src_hbm.at[pl.ds(off, _CH)], sidx)
        pltpu.sync_copy(dst_hbm.at[pl.ds(off, _CH)], didx)
        pltpu.async_copy(xs_hbm.at[c].at[sidx], rows, sem).wait()
        pltpu.sync_copy(rows, table.at[didx], add=True)
    _fori(_FULL, chunk)

    off = base + _FULL * _CH
    pltpu.sync_copy(src_hbm.at[pl.ds(off, _REM)], sidx2)
    pltpu.sync_copy(dst_hbm.at[pl.ds(off, _REM)], didx2)
    pltpu.async_copy(xs_hbm.at[c].at[sidx2], rows2, sem).wait()
    pltpu.sync_copy(rows2, table.at[didx2], add=True)

    plsc.subcore_barrier()

    @pl.when(s < _NS - 1)
    def _out_main():
        pltpu.sync_copy(table.at[pl.ds(r0, _RPT)],
                        u_out.at[c, pl.ds(r0, _RPT)])

    @pl.when(s == _NS - 1)
    def _out_last():
        pltpu.sync_copy(table.at[pl.ds((_NS - 1) * _RPT, _RPT_LAST)],
                        u_out.at[c, pl.ds((_NS - 1) * _RPT, _RPT_LAST)])


_sc_pass2 = functools.partial(
    pl.kernel,
    out_type=jax.ShapeDtypeStruct((_NC, _N, _D), jnp.float32),
    mesh=plsc.VectorSubcoreMesh(core_axis_name="c", subcore_axis_name="s",
                                num_cores=_NC, num_subcores=_NS),
    scratch_types=[
        pltpu.VMEM_SHARED((_N, _D), jnp.float32),
        pltpu.VMEM((_CH,), jnp.int32),
        pltpu.VMEM((_CH,), jnp.int32),
        pltpu.VMEM((_CH, _D), jnp.float32),
        pltpu.VMEM((_REM,), jnp.int32),
        pltpu.VMEM((_REM,), jnp.int32),
        pltpu.VMEM((_REM, _D), jnp.float32),
        pltpu.SemaphoreType.DMA,
    ],
)(_sc_pass2_body)


_BLK = 1000
_GRID = _N // _BLK


def _tc_scale_body(deg_ref, feat_ref, xperm_ref, dis_ref, xs_ref):
    dis = lax.rsqrt(deg_ref[...] + 1.0)
    dis_ref[...] = dis
    xs_ref[0] = feat_ref[...] * dis
    xs_ref[1] = xperm_ref[...] * dis


def _tc_scale(deg, feat, xperm):
    return pl.pallas_call(
        _tc_scale_body,
        grid=(_GRID,),
        in_specs=[
            pl.BlockSpec((_BLK, 1), lambda i: (i, 0)),
            pl.BlockSpec((_BLK, _D), lambda i: (i, 0)),
            pl.BlockSpec((_BLK, _D), lambda i: (i, 0)),
        ],
        out_specs=[
            pl.BlockSpec((_BLK, 1), lambda i: (i, 0)),
            pl.BlockSpec((_NC, _BLK, _D), lambda i: (0, i, 0)),
        ],
        out_shape=[
            jax.ShapeDtypeStruct((_N, 1), jnp.float32),
            jax.ShapeDtypeStruct((_NC, _N, _D), jnp.float32),
        ],
    )(deg, feat, xperm)


def _tc_b1_body(u_ref, dis_ref, w_ref, b_ref, adjt_ref,
                pos_ref, neg_ref, pooled_ref):
    i = pl.program_id(0)
    dis = dis_ref[...]
    w = w_ref[...]
    b = b_ref[...]
    pos = jnp.maximum(
        jnp.dot(u_ref[0] * dis, w, preferred_element_type=jnp.float32) + b, 0.0)
    neg = jnp.maximum(
        jnp.dot(u_ref[1] * dis, w, preferred_element_type=jnp.float32) + b, 0.0)
    pos_ref[...] = pos
    neg_ref[...] = neg
    contrib = lax.dot_general(adjt_ref[...], pos, (((0,), (0,)), ((), ())),
                              preferred_element_type=jnp.float32)

    @pl.when(i == 0)
    def _init():
        pooled_ref[...] = contrib

    @pl.when(i != 0)
    def _acc():
        pooled_ref[...] = pooled_ref[...] + contrib


def _tc_b1(u, dis, w, b, adjt):
    return pl.pallas_call(
        _tc_b1_body,
        grid=(_GRID,),
        in_specs=[
            pl.BlockSpec((_NC, _BLK, _D), lambda i: (0, i, 0)),
            pl.BlockSpec((_BLK, 1), lambda i: (i, 0)),
            pl.BlockSpec((_D, _D), lambda i: (0, 0)),
            pl.BlockSpec((1, _D), lambda i: (0, 0)),
            pl.BlockSpec((_BLK, _S), lambda i: (i, 0)),
        ],
        out_specs=[
            pl.BlockSpec((_BLK, _D), lambda i: (i, 0)),
            pl.BlockSpec((_BLK, _D), lambda i: (i, 0)),
            pl.BlockSpec((_S, _D), lambda i: (0, 0)),
        ],
        out_shape=[
            jax.ShapeDtypeStruct((_N, _D), jnp.float32),
            jax.ShapeDtypeStruct((_N, _D), jnp.float32),
            jax.ShapeDtypeStruct((_S, _D), jnp.float32),
        ],
    )(u, dis, w, b, adjt)


def _tc_b2_body(pooled_ref, norm_ref, adjt_ref, pos_ref, neg_ref, wd_ref,
                out_ref, gr_scr, acc_scr):
    i = pl.program_id(0)

    @pl.when(i == 0)
    def _init():
        ge = pooled_ref[...] / norm_ref[...]
        gr_scr[...] = 1.0 / (1.0 + jnp.exp(-ge))
        acc_scr[0] = 0.0
        acc_scr[1] = 0.0

    summary = jnp.dot(adjt_ref[...], gr_scr[...],
                      preferred_element_type=jnp.float32)
    wd = wd_ref[...]
    pw = jnp.dot(pos_ref[...], wd, preferred_element_type=jnp.float32)
    nw = jnp.dot(neg_ref[...], wd, preferred_element_type=jnp.float32)
    pos_logits = jnp.sum(pw * summary, axis=1)
    neg_logits = jnp.sum(nw * summary, axis=1)
    pos_terms = (jnp.maximum(pos_logits, 0.0) - pos_logits
                 + jnp.log1p(jnp.exp(-jnp.abs(pos_logits))))
    neg_terms = (jnp.maximum(neg_logits, 0.0)
                 + jnp.log1p(jnp.exp(-jnp.abs(neg_logits))))
    acc_scr[0] = acc_scr[0] + jnp.sum(pos_terms)
    acc_scr[1] = acc_scr[1] + jnp.sum(neg_terms)

    @pl.when(i == pl.num_programs(0) - 1)
    def _fin():
        out_ref[...] = (jnp.stack([acc_scr[0], acc_scr[1]])
                        .reshape(1, 2) / _N)


def _tc_b2(pooled, norm, adjt, pos, neg, wd):
    return pl.pallas_call(
        _tc_b2_body,
        grid=(_GRID,),
        in_specs=[
            pl.BlockSpec((_S, _D), lambda i: (0, 0)),
            pl.BlockSpec((_S, 1), lambda i: (0, 0)),
            pl.BlockSpec((_BLK, _S), lambda i: (i, 0)),
            pl.BlockSpec((_BLK, _D), lambda i: (i, 0)),
            pl.BlockSpec((_BLK, _D), lambda i: (i, 0)),
            pl.BlockSpec((_D, _D), lambda i: (0, 0)),
        ],
        out_specs=pl.BlockSpec((1, 2), lambda i: (0, 0)),
        out_shape=jax.ShapeDtypeStruct((1, 2), jnp.float32),
        scratch_shapes=[
            pltpu.VMEM((_S, _D), jnp.float32),
            pltpu.SMEM((2,), jnp.float32),
        ],
    )(pooled, norm, adjt, pos, neg, wd)


def kernel(features, edge_index, subgraph_adj, subgraph_norm, node_subgraph,
           node_list, perm, W_gcn, b_gcn, W_disc):
    src = edge_index[0].astype(jnp.int32)
    dst = edge_index[1].astype(jnp.int32)
    perm32 = perm.astype(jnp.int32)

    deg_pad, xperm = _sc_pass1(dst, perm32, features)
    deg = deg_pad[:_N, :1]
    dis, xs = _tc_scale(deg, features, xperm)
    u = _sc_pass2(xs, src, dst)
    adjt = subgraph_adj.T
    pos, neg, pooled = _tc_b1(u, dis, W_gcn, b_gcn.reshape(1, _D), adjt)
    out = _tc_b2(pooled, subgraph_norm, adjt, pos, neg, W_disc)
    return (out[0, 0], out[0, 1])
